# Initial kernel scaffold; baseline (speedup 1.0000x reference)
import jax
import jax.numpy as jnp
from jax.experimental import pallas as pl


def kernel(coords, feats, W, bn_gamma, bn_beta, bn_mean, bn_var):
    raise NotImplementedError("write your pallas kernel here")


# speedup vs baseline: 4.5392x; 4.5392x over previous
import jax
import jax.numpy as jnp
from jax.experimental import pallas as pl
from jax.experimental.pallas import tpu as pltpu

RADIUS = 0.1
NSAMPLE = 32
DENSITY_K = 8
EPS = 1e-08
BN_EPS = 1e-05

N = 2048
C = 128
RB = 256
HIGHEST = jax.lax.Precision.HIGHEST


def _proj_kernel(feats_ref, wf_ref, g_ref):
    f = feats_ref[0]
    wf = wf_ref[...]
    g_ref[0] = jax.lax.dot_general(
        f, wf, (((0,), (1,)), ((), ())), preferred_element_type=jnp.float32)


def _agg_kernel(cblk_ref, cfull_ref, ctr_ref, gfull_ref, wrp_ref, scale_ref,
                shift_ref, out_ref, d2_ref, msg_ref):
    cb = cblk_ref[0]
    cf = cfull_ref[0]
    ct = ctr_ref[0]
    gf = gfull_ref[0]
    wrp = wrp_ref[...]
    scale = scale_ref[...]
    shift = shift_ref[...]

    cbx = cb[:, 0:1]
    cby = cb[:, 1:2]
    cbz = cb[:, 2:3]
    nb = cbx * cbx + cby * cby + cbz * cbz
    nf = (ct[0:1, :] * ct[0:1, :] + ct[1:2, :] * ct[1:2, :]
          + ct[2:3, :] * ct[2:3, :])
    cross = jax.lax.dot_general(cb, cf, (((1,), (1,)), ((), ())),
                                preferred_element_type=jnp.float32)
    d2_ref[...] = jnp.clip((nb + nf) - 2.0 * cross, 1e-12, None)

    iota_n = jax.lax.broadcasted_iota(jnp.int32, (RB, N), 1).astype(jnp.float32)
    lane32 = jax.lax.broadcasted_iota(jnp.int32, (RB, NSAMPLE), 1)

    def body(k, carry):
        gx, gy, gz = carry
        d2 = d2_ref[...]
        m = jnp.min(d2, axis=1, keepdims=True)
        amin = jnp.min(jnp.where(d2 == m, iota_n, float(N)),
                       axis=1, keepdims=True)
        sel = (iota_n == amin).astype(jnp.float32)
        d2_ref[...] = jnp.where(iota_n == amin, jnp.inf, d2)
        gc = jax.lax.dot_general(sel, cf, (((1,), (0,)), ((), ())),
                                 preferred_element_type=jnp.float32,
                                 precision=HIGHEST)
        grow = jax.lax.dot_general(sel, gf, (((1,), (0,)), ((), ())),
                                   preferred_element_type=jnp.float32)
        gxk = gc[:, 0:1]
        gyk = gc[:, 1:2]
        gzk = gc[:, 2:3]
        rxk = (gxk - cbx) * (1.0 / RADIUS)
        ryk = (gyk - cby) * (1.0 / RADIUS)
        rzk = (gzk - cbz) * (1.0 / RADIUS)
        relproj = rxk * wrp[0:1, :] + ryk * wrp[1:2, :] + rzk * wrp[2:3, :]
        msg_ref[k] = jnp.maximum((grow + relproj) * scale + shift, 0.0)
        gx = jnp.where(lane32 == k, gxk, gx)
        gy = jnp.where(lane32 == k, gyk, gy)
        gz = jnp.where(lane32 == k, gzk, gz)
        return gx, gy, gz

    zero = jnp.zeros((RB, NSAMPLE), jnp.float32)
    gx, gy, gz = jax.lax.fori_loop(0, NSAMPLE, body, (zero, zero, zero))

    ni = gx * gx + gy * gy + gz * gz
    gx16 = gx.astype(jnp.bfloat16).astype(jnp.float32)
    gy16 = gy.astype(jnp.bfloat16).astype(jnp.float32)
    gz16 = gz.astype(jnp.bfloat16).astype(jnp.float32)
    crossp = ((gx16[:, :, None] * gx16[:, None, :]
               + gy16[:, :, None] * gy16[:, None, :])
              + gz16[:, :, None] * gz16[:, None, :])
    pd = jnp.clip((ni[:, :, None] + ni[:, None, :]) - 2.0 * crossp,
                  1e-12, None)
    jjj = jax.lax.broadcasted_iota(jnp.int32, (RB, NSAMPLE, NSAMPLE), 1)
    iii = jax.lax.broadcasted_iota(jnp.int32, (RB, NSAMPLE, NSAMPLE), 2)
    pd = jnp.where(jjj == iii, jnp.inf, pd)
    iota_j = jjj.astype(jnp.float32)

    def dbody(_, pdc):
        m = jnp.min(pdc, axis=1, keepdims=True)
        am = jnp.min(jnp.where(pdc == m, iota_j, float(NSAMPLE)),
                     axis=1, keepdims=True)
        return jnp.where(iota_j == am, jnp.inf, pdc)

    pd = jax.lax.fori_loop(0, DENSITY_K - 1, dbody, pd)
    kth_d2 = jnp.min(pd, axis=1)
    kth = jnp.sqrt(kth_d2)
    raw = jnp.clip(kth, EPS, None)
    raw = raw * raw * raw
    w = raw / jnp.clip(jnp.sum(raw, axis=1, keepdims=True), EPS, None)

    acc = jnp.zeros((RB, C), jnp.float32)
    for k in range(NSAMPLE):
        acc = acc + msg_ref[k] * w[:, k:k + 1]

    ri = jax.lax.broadcasted_iota(jnp.int32, (RB, RB), 0)
    ci = jax.lax.broadcasted_iota(jnp.int32, (RB, RB), 1)
    eye = (ri == ci).astype(jnp.float32)
    out_ref[0] = jax.lax.dot_general(acc, eye, (((0,), (0,)), ((), ())),
                                     preferred_element_type=jnp.float32,
                                     precision=HIGHEST)


@jax.jit
def kernel(coords, feats, W, bn_gamma, bn_beta, bn_mean, bn_var):
    B = coords.shape[0]
    scale = (bn_gamma / jnp.sqrt(bn_var + BN_EPS))[None, :]
    shift = (bn_beta - bn_mean * bn_gamma / jnp.sqrt(bn_var + BN_EPS))[None, :]
    wrp = W[:, :3].T
    wfp = W[:, 3:]
    coords_t = jnp.swapaxes(coords, 1, 2)

    g = pl.pallas_call(
        _proj_kernel,
        grid=(B,),
        in_specs=[
            pl.BlockSpec((1, C, N), lambda b: (b, 0, 0)),
            pl.BlockSpec((C, C), lambda b: (0, 0)),
        ],
        out_specs=pl.BlockSpec((1, N, C), lambda b: (b, 0, 0)),
        out_shape=jax.ShapeDtypeStruct((B, N, C), jnp.float32),
    )(feats, wfp)

    nblk = N // RB
    out = pl.pallas_call(
        _agg_kernel,
        grid=(B, nblk),
        in_specs=[
            pl.BlockSpec((1, RB, 3), lambda b, r: (b, r, 0)),
            pl.BlockSpec((1, N, 3), lambda b, r: (b, 0, 0)),
            pl.BlockSpec((1, 3, N), lambda b, r: (b, 0, 0)),
            pl.BlockSpec((1, N, C), lambda b, r: (b, 0, 0)),
            pl.BlockSpec((3, C), lambda b, r: (0, 0)),
            pl.BlockSpec((1, C), lambda b, r: (0, 0)),
            pl.BlockSpec((1, C), lambda b, r: (0, 0)),
        ],
        out_specs=pl.BlockSpec((1, C, RB), lambda b, r: (b, 0, r)),
        out_shape=jax.ShapeDtypeStruct((B, C, N), jnp.float32),
        scratch_shapes=[
            pltpu.VMEM((RB, N), jnp.float32),
            pltpu.VMEM((NSAMPLE, RB, C), jnp.float32),
        ],
    )(coords, coords, coords_t, g, wrp, scale, shift)
    return out


# speedup vs baseline: 8.2602x; 1.8197x over previous
import jax
import jax.numpy as jnp
from jax.experimental import pallas as pl
from jax.experimental.pallas import tpu as pltpu

RADIUS = 0.1
NSAMPLE = 32
DENSITY_K = 8
EPS = 1e-08
BN_EPS = 1e-05

N = 2048
C = 128
RB = 256
HIGHEST = jax.lax.Precision.HIGHEST


def _tmin0(x):
    s = x.shape[0]
    while s > 8:
        h = s // 2
        x = jnp.minimum(x[:h], x[h:s])
        s = h
    return jnp.min(x, axis=0, keepdims=True)


def _tsum0(x):
    s = x.shape[0]
    while s > 8:
        h = s // 2
        x = x[:h] + x[h:s]
        s = h
    return jnp.sum(x, axis=0, keepdims=True)


def _table_kernel(feats_ref, coords_ref, wf_ref, wrp_ref, scale_ref, t_ref):
    f = feats_ref[0]
    cf = coords_ref[0]
    wf = wf_ref[...]
    wrp = wrp_ref[...]
    scale = scale_ref[...]
    g = jax.lax.dot_general(f, wf, (((0,), (1,)), ((), ())),
                            preferred_element_type=jnp.float32)
    cfw = jax.lax.dot_general(cf * 10.0, wrp, (((1,), (0,)), ((), ())),
                              preferred_element_type=jnp.float32)
    t_ref[0] = (g + cfw) * scale


def _agg_kernel(ctb_ref, cfull_ref, t_ref, wrp_ref, scalet_ref, shiftt_ref,
                out_ref, d2_ref, msg_ref, gct_ref):
    ctb = ctb_ref[0]
    cf = cfull_ref[0]
    tb = t_ref[0]
    wrp = wrp_ref[...]
    scalet = scalet_ref[...]
    shiftt = shiftt_ref[...]

    nbt = (ctb[0:1, :] * ctb[0:1, :] + ctb[1:2, :] * ctb[1:2, :]
           + ctb[2:3, :] * ctb[2:3, :])
    cfx = cf[:, 0:1]
    cfy = cf[:, 1:2]
    cfz = cf[:, 2:3]
    nft = cfx * cfx + cfy * cfy + cfz * cfz
    crosst = jax.lax.dot_general(cf, ctb, (((1,), (0,)), ((), ())),
                                 preferred_element_type=jnp.float32)
    d2_ref[...] = jnp.clip((nbt + nft) - 2.0 * crosst, 1e-12, None)

    iota0 = jax.lax.broadcasted_iota(jnp.int32, (N, RB), 0).astype(jnp.float32)
    cbw = jax.lax.dot_general(wrp, ctb * 10.0, (((0,), (0,)), ((), ())),
                              preferred_element_type=jnp.float32)
    negct = shiftt - cbw * scalet

    def body(k, _):
        d2 = d2_ref[...]
        m = _tmin0(d2)
        amin = _tmin0(jnp.where(d2 == m, iota0, float(N)))
        selt = (iota0 == amin).astype(jnp.float32)
        d2_ref[...] = jnp.where(iota0 == amin, jnp.inf, d2)
        gct = jax.lax.dot_general(cf, selt, (((0,), (0,)), ((), ())),
                                  preferred_element_type=jnp.float32,
                                  precision=HIGHEST)
        msgt = jax.lax.dot_general(tb, selt, (((0,), (0,)), ((), ())),
                                   preferred_element_type=jnp.float32)
        msg_ref[k] = jnp.maximum(msgt + negct, 0.0)
        gct_ref[k] = gct
        return 0

    jax.lax.fori_loop(0, NSAMPLE, body, 0)

    gxt = gct_ref[:, 0, :]
    gyt = gct_ref[:, 1, :]
    gzt = gct_ref[:, 2, :]
    nit = gxt * gxt + gyt * gyt + gzt * gzt
    gx16 = gxt.astype(jnp.bfloat16).astype(jnp.float32)
    gy16 = gyt.astype(jnp.bfloat16).astype(jnp.float32)
    gz16 = gzt.astype(jnp.bfloat16).astype(jnp.float32)
    crossp = ((gx16[:, None, :] * gx16[None, :, :]
               + gy16[:, None, :] * gy16[None, :, :])
              + gz16[:, None, :] * gz16[None, :, :])
    pd = jnp.clip((nit[:, None, :] + nit[None, :, :]) - 2.0 * crossp,
                  1e-12, None)
    jjj = jax.lax.broadcasted_iota(jnp.int32, (NSAMPLE, NSAMPLE, RB), 0)
    iii = jax.lax.broadcasted_iota(jnp.int32, (NSAMPLE, NSAMPLE, RB), 1)
    pd = jnp.where(jjj == iii, jnp.inf, pd)
    iota_j = jjj.astype(jnp.float32)

    def dbody(_, pdc):
        m = _tmin0(pdc)
        am = _tmin0(jnp.where(pdc == m, iota_j, float(NSAMPLE)))
        return jnp.where(iota_j == am, jnp.inf, pdc)

    pd = jax.lax.fori_loop(0, DENSITY_K - 1, dbody, pd)
    kth_d2 = _tmin0(pd).reshape(NSAMPLE, RB)
    kth = jnp.sqrt(kth_d2)
    raw = jnp.clip(kth, EPS, None)
    raw = raw * raw * raw
    w = raw / jnp.clip(_tsum0(raw), EPS, None)

    acc = jnp.zeros((C, RB), jnp.float32)
    for k in range(NSAMPLE):
        acc = acc + msg_ref[k] * w[k:k + 1, :]
    out_ref[0] = acc


@jax.jit
def kernel(coords, feats, W, bn_gamma, bn_beta, bn_mean, bn_var):
    B = coords.shape[0]
    scale = bn_gamma / jnp.sqrt(bn_var + BN_EPS)
    shift = bn_beta - bn_mean * scale
    wrp = W[:, :3].T
    wfp = W[:, 3:]
    coords_t = jnp.swapaxes(coords, 1, 2)

    tbl = pl.pallas_call(
        _table_kernel,
        grid=(B,),
        in_specs=[
            pl.BlockSpec((1, C, N), lambda b: (b, 0, 0)),
            pl.BlockSpec((1, N, 3), lambda b: (b, 0, 0)),
            pl.BlockSpec((C, C), lambda b: (0, 0)),
            pl.BlockSpec((3, C), lambda b: (0, 0)),
            pl.BlockSpec((1, C), lambda b: (0, 0)),
        ],
        out_specs=pl.BlockSpec((1, N, C), lambda b: (b, 0, 0)),
        out_shape=jax.ShapeDtypeStruct((B, N, C), jnp.float32),
    )(feats, coords, wfp, wrp, scale[None, :])

    nblk = N // RB
    out = pl.pallas_call(
        _agg_kernel,
        grid=(B, nblk),
        in_specs=[
            pl.BlockSpec((1, 3, RB), lambda b, r: (b, 0, r)),
            pl.BlockSpec((1, N, 3), lambda b, r: (b, 0, 0)),
            pl.BlockSpec((1, N, C), lambda b, r: (b, 0, 0)),
            pl.BlockSpec((3, C), lambda b, r: (0, 0)),
            pl.BlockSpec((C, 1), lambda b, r: (0, 0)),
            pl.BlockSpec((C, 1), lambda b, r: (0, 0)),
        ],
        out_specs=pl.BlockSpec((1, C, RB), lambda b, r: (b, 0, r)),
        out_shape=jax.ShapeDtypeStruct((B, C, N), jnp.float32),
        scratch_shapes=[
            pltpu.VMEM((N, RB), jnp.float32),
            pltpu.VMEM((NSAMPLE, C, RB), jnp.float32),
            pltpu.VMEM((NSAMPLE, 3, RB), jnp.float32),
        ],
    )(coords_t, coords, tbl, wrp, scale[:, None], shift[:, None])
    return out
